# Initial kernel scaffold; baseline (speedup 1.0000x reference)
import functools

import jax
import jax.numpy as jnp
from jax import lax
from jax.experimental import pallas as pl
from jax.experimental.pallas import tpu as pltpu
from jax.experimental.pallas import tpu_sc as plsc

N = 10000
N0 = 10000
M = 32
D_IN = 128
D_OUT = 128
K = 15
POINT_INFLUENCE = 0.05

NW = 32
E = N * M
EW = E // NW
CH = 400
NCH = EW // CH

BQ = 200
EB = BQ * M
GRID = N // BQ


@functools.partial(
    pl.kernel,
    out_type=[
        jax.ShapeDtypeStruct((E, D_IN), jnp.float32),
        jax.ShapeDtypeStruct((E, 16), jnp.float32),
    ],
    mesh=plsc.VectorSubcoreMesh(core_axis_name="c", subcore_axis_name="s"),
    scratch_types=[
        pltpu.VMEM((CH,), jnp.int32),
        pltpu.VMEM((CH, D_IN), jnp.float32),
        pltpu.VMEM((CH, 16), jnp.float32),
        pltpu.SemaphoreType.DMA,
        pltpu.SemaphoreType.DMA,
    ],
)
def _sc_gather(nb_hbm, x_hbm, sp16_hbm, g_out, p_out, idx_v, gbuf, pbuf, sem1, sem2):
    wid = lax.axis_index("s") * 2 + lax.axis_index("c")
    base = wid * EW

    def body(i, carry):
        off = pl.multiple_of(base + i * CH, 8)
        pltpu.sync_copy(nb_hbm.at[pl.ds(off, CH)], idx_v)
        cp1 = pltpu.async_copy(x_hbm.at[idx_v], gbuf, sem1)
        cp2 = pltpu.async_copy(sp16_hbm.at[idx_v], pbuf, sem2)
        cp1.wait()
        cp2.wait()
        pltpu.sync_copy(gbuf, g_out.at[pl.ds(off, CH)])
        pltpu.sync_copy(pbuf, p_out.at[pl.ds(off, CH)])
        return carry

    lax.fori_loop(0, NCH, body, 0)


def _tc_body(g_ref, p_ref, q_ref, kpt_ref, kpsq_ref, w_ref, o_ref):
    q = q_ref[...]
    qe = jnp.broadcast_to(q.reshape(BQ, 1, 16), (BQ, M, 16)).reshape(EB, 16)
    d = p_ref[...] - qe
    dd = jnp.sum(d * d, axis=1, keepdims=True)
    cross = lax.dot_general(
        d, kpt_ref[...], (((1,), (0,)), ((), ())),
        precision=lax.Precision.HIGHEST,
        preferred_element_type=jnp.float32,
    )
    sq = jnp.maximum(dd - 2.0 * cross + kpsq_ref[...], 0.0)
    a = jnp.maximum(1.0 - jnp.sqrt(sq) * (1.0 / POINT_INFLUENCE), 0.0)
    a3 = a.reshape(BQ, M, 16)
    g3 = g_ref[...].reshape(BQ, M, D_IN)
    cols = [jnp.sum(a3[:, :, k:k + 1] * g3, axis=1) for k in range(K)]
    wf = jnp.concatenate(cols, axis=1)
    o_ref[...] = lax.dot_general(
        wf, w_ref[...], (((1,), (0,)), ((), ())),
        preferred_element_type=jnp.float32,
    )


def kernel(query_points, support_points, neighbors, x, K_points, weight):
    sp16 = jnp.pad(support_points, ((0, 0), (0, 13)))
    q16 = jnp.pad(query_points, ((0, 0), (0, 13)))
    nbf = neighbors.reshape(-1)
    g, p16 = _sc_gather(nbf, x, sp16)
    kpt = jnp.pad(K_points.T, ((0, 13), (0, 1)))
    kpsq = jnp.pad(jnp.sum(K_points * K_points, axis=1)[None, :],
                   ((0, 0), (0, 1)))
    wflat = weight.reshape(K * D_IN, D_OUT)

    out = pl.pallas_call(
        _tc_body,
        grid=(GRID,),
        in_specs=[
            pl.BlockSpec((EB, D_IN), lambda i: (i, 0)),
            pl.BlockSpec((EB, 16), lambda i: (i, 0)),
            pl.BlockSpec((BQ, 16), lambda i: (i, 0)),
            pl.BlockSpec((16, 16), lambda i: (0, 0)),
            pl.BlockSpec((1, 16), lambda i: (0, 0)),
            pl.BlockSpec((K * D_IN, D_OUT), lambda i: (0, 0)),
        ],
        out_specs=pl.BlockSpec((BQ, D_OUT), lambda i: (i, 0)),
        out_shape=jax.ShapeDtypeStruct((N, D_OUT), jnp.float32),
    )(g, p16, q16, kpt, kpsq, wflat)
    return out


# speedup vs baseline: 1.3681x; 1.3681x over previous
import functools

import jax
import jax.numpy as jnp
from jax import lax
from jax.experimental import pallas as pl
from jax.experimental.pallas import tpu as pltpu
from jax.experimental.pallas import tpu_sc as plsc

N = 10000
N0 = 10000
M = 32
D_IN = 128
D_OUT = 128
K = 15
POINT_INFLUENCE = 0.05

NW = 32
E = N * M
EW = E // NW
CH = 400
NCH = EW // CH

BQ = 200
EB = BQ * M
GRID = N // BQ


@functools.partial(
    pl.kernel,
    out_type=[
        jax.ShapeDtypeStruct((E, D_IN), jnp.float32),
        jax.ShapeDtypeStruct((E, 16), jnp.float32),
    ],
    mesh=plsc.VectorSubcoreMesh(core_axis_name="c", subcore_axis_name="s"),
    compiler_params=pltpu.CompilerParams(use_tc_tiling_on_sc=False),
    scratch_types=[
        pltpu.VMEM((CH,), jnp.int32),
        pltpu.VMEM((CH, D_IN), jnp.float32),
        pltpu.VMEM((CH, 16), jnp.float32),
        pltpu.SemaphoreType.DMA,
        pltpu.SemaphoreType.DMA,
    ],
)
def _sc_gather(nb_hbm, x_hbm, sp16_hbm, g_out, p_out, idx_v, gbuf, pbuf, sem1, sem2):
    wid = lax.axis_index("s") * 2 + lax.axis_index("c")
    base = wid * EW

    def body(i, carry):
        off = pl.multiple_of(base + i * CH, 8)
        pltpu.sync_copy(nb_hbm.at[pl.ds(off, CH)], idx_v)
        cp1 = pltpu.async_copy(x_hbm.at[idx_v], gbuf, sem1)
        cp2 = pltpu.async_copy(sp16_hbm.at[idx_v], pbuf, sem2)
        cp1.wait()
        cp2.wait()
        pltpu.sync_copy(gbuf, g_out.at[pl.ds(off, CH)])
        pltpu.sync_copy(pbuf, p_out.at[pl.ds(off, CH)])
        return carry

    lax.fori_loop(0, NCH, body, 0)


def _tc_body(g_ref, p_ref, q_ref, kpt_ref, kpsq_ref, w_ref, o_ref):
    q = q_ref[...]
    qe = jnp.broadcast_to(q.reshape(BQ, 1, 16), (BQ, M, 16)).reshape(EB, 16)
    d = p_ref[...] - qe
    dd = jnp.sum(d * d, axis=1, keepdims=True)
    cross = lax.dot_general(
        d, kpt_ref[...], (((1,), (0,)), ((), ())),
        precision=lax.Precision.HIGHEST,
        preferred_element_type=jnp.float32,
    )
    sq = jnp.maximum(dd - 2.0 * cross + kpsq_ref[...], 0.0)
    a = jnp.maximum(1.0 - jnp.sqrt(sq) * (1.0 / POINT_INFLUENCE), 0.0)
    a3 = a.reshape(BQ, M, 16)
    g3 = g_ref[...].reshape(BQ, M, D_IN)
    cols = [jnp.sum(a3[:, :, k:k + 1] * g3, axis=1) for k in range(K)]
    wf = jnp.concatenate(cols, axis=1)
    o_ref[...] = lax.dot_general(
        wf, w_ref[...], (((1,), (0,)), ((), ())),
        preferred_element_type=jnp.float32,
    )


def kernel(query_points, support_points, neighbors, x, K_points, weight):
    sp16 = jnp.pad(support_points, ((0, 0), (0, 13)))
    q16 = jnp.pad(query_points, ((0, 0), (0, 13)))
    nbf = neighbors.reshape(-1)
    g, p16 = _sc_gather(nbf, x, sp16)
    kpt = jnp.pad(K_points.T, ((0, 13), (0, 1)))
    kpsq = jnp.pad(jnp.sum(K_points * K_points, axis=1)[None, :],
                   ((0, 0), (0, 1)))
    wflat = weight.reshape(K * D_IN, D_OUT)

    out = pl.pallas_call(
        _tc_body,
        grid=(GRID,),
        in_specs=[
            pl.BlockSpec((EB, D_IN), lambda i: (i, 0)),
            pl.BlockSpec((EB, 16), lambda i: (i, 0)),
            pl.BlockSpec((BQ, 16), lambda i: (i, 0)),
            pl.BlockSpec((16, 16), lambda i: (0, 0)),
            pl.BlockSpec((1, 16), lambda i: (0, 0)),
            pl.BlockSpec((K * D_IN, D_OUT), lambda i: (0, 0)),
        ],
        out_specs=pl.BlockSpec((BQ, D_OUT), lambda i: (i, 0)),
        out_shape=jax.ShapeDtypeStruct((N, D_OUT), jnp.float32),
    )(g, p16, q16, kpt, kpsq, wflat)
    return out


# speedup vs baseline: 1.9162x; 1.4007x over previous
import functools

import jax
import jax.numpy as jnp
from jax import lax
from jax.experimental import pallas as pl
from jax.experimental.pallas import tpu as pltpu
from jax.experimental.pallas import tpu_sc as plsc

N = 10000
N0 = 10000
M = 32
D_IN = 128
D_OUT = 128
K = 15
POINT_INFLUENCE = 0.05

NW = 32
E = N * M
EW = E // NW
CH = 400
NCH = EW // CH

BQ = 200
EB = BQ * M
GRID = N // BQ
GQ = 8
NG = BQ // GQ
GE = GQ * M


@functools.partial(
    pl.kernel,
    out_type=[
        jax.ShapeDtypeStruct((E, D_IN), jnp.float32),
        jax.ShapeDtypeStruct((E, 16), jnp.float32),
    ],
    mesh=plsc.VectorSubcoreMesh(core_axis_name="c", subcore_axis_name="s"),
    compiler_params=pltpu.CompilerParams(use_tc_tiling_on_sc=False),
    scratch_types=[
        pltpu.VMEM((CH,), jnp.int32),
        pltpu.VMEM((CH, D_IN), jnp.float32),
        pltpu.VMEM((CH, 16), jnp.float32),
        pltpu.SemaphoreType.DMA,
        pltpu.SemaphoreType.DMA,
    ],
)
def _sc_gather(nb_hbm, x_hbm, sp16_hbm, g_out, p_out, idx_v, gbuf, pbuf, sem1, sem2):
    wid = lax.axis_index("s") * 2 + lax.axis_index("c")
    base = wid * EW

    def body(i, carry):
        off = pl.multiple_of(base + i * CH, 8)
        pltpu.sync_copy(nb_hbm.at[pl.ds(off, CH)], idx_v)
        cp1 = pltpu.async_copy(x_hbm.at[idx_v], gbuf, sem1)
        cp2 = pltpu.async_copy(sp16_hbm.at[idx_v], pbuf, sem2)
        cp1.wait()
        cp2.wait()
        pltpu.sync_copy(gbuf, g_out.at[pl.ds(off, CH)])
        pltpu.sync_copy(pbuf, p_out.at[pl.ds(off, CH)])
        return carry

    lax.fori_loop(0, NCH, body, 0)


def _tc_body(g_ref, p_ref, qr_ref, kpt_ref, ones_ref, kpsq_ref, rep_ref,
             mask_ref, w_ref, o_ref):
    d = p_ref[...] - qr_ref[...]
    cross = lax.dot_general(
        d, kpt_ref[...], (((1,), (0,)), ((), ())),
        precision=lax.Precision.HIGHEST,
        preferred_element_type=jnp.float32,
    )
    ddr = lax.dot_general(
        d * d, ones_ref[...], (((1,), (0,)), ((), ())),
        precision=lax.Precision.HIGHEST,
        preferred_element_type=jnp.float32,
    )
    sq = jnp.maximum(ddr - 2.0 * cross + kpsq_ref[...], 0.0)
    a = jnp.maximum(1.0 - jnp.sqrt(sq) * (1.0 / POINT_INFLUENCE), 0.0)
    arep = lax.dot_general(
        a, rep_ref[...], (((1,), (0,)), ((), ())),
        preferred_element_type=jnp.float32,
    )
    bd = arep * mask_ref[...]
    g = g_ref[...]
    wfs = []
    for grp in range(NG):
        wfs.append(lax.dot_general(
            bd[grp * GE:(grp + 1) * GE, :], g[grp * GE:(grp + 1) * GE, :],
            (((0,), (0,)), ((), ())),
            preferred_element_type=jnp.float32,
        ))
    wf3 = jnp.concatenate(wfs, axis=0).reshape(NG, 128, D_IN)
    acc = jnp.zeros((BQ, D_OUT), jnp.float32)
    for k in range(K):
        wk = wf3[:, k * GQ:(k + 1) * GQ, :].reshape(BQ, D_IN)
        acc = acc + lax.dot_general(
            wk, w_ref[k * D_IN:(k + 1) * D_IN, :], (((1,), (0,)), ((), ())),
            preferred_element_type=jnp.float32,
        )
    o_ref[...] = acc


def kernel(query_points, support_points, neighbors, x, K_points, weight):
    sp16 = jnp.pad(support_points, ((0, 0), (0, 13)))
    q16 = jnp.pad(query_points, ((0, 0), (0, 13)))
    qrep = jnp.repeat(q16, M, axis=0)
    nbf = neighbors.reshape(-1)
    g, p16 = _sc_gather(nbf, x, sp16)
    kpt = jnp.pad(K_points.T, ((0, 13), (0, 1)))
    ones16 = jnp.ones((16, 16), jnp.float32)
    kpsq = jnp.pad(jnp.sum(K_points * K_points, axis=1)[None, :],
                   ((0, 0), (0, 1)), constant_values=1e6)
    rep = (jnp.arange(128)[None, :] // GQ
           == jnp.arange(16)[:, None]).astype(jnp.float32)
    mask = (jnp.arange(128)[None, :] % GQ
            == (jnp.arange(EB) // M % GQ)[:, None]).astype(jnp.float32)
    wflat = weight.reshape(K * D_IN, D_OUT)

    out = pl.pallas_call(
        _tc_body,
        grid=(GRID,),
        in_specs=[
            pl.BlockSpec((EB, D_IN), lambda i: (i, 0)),
            pl.BlockSpec((EB, 16), lambda i: (i, 0)),
            pl.BlockSpec((EB, 16), lambda i: (i, 0)),
            pl.BlockSpec((16, 16), lambda i: (0, 0)),
            pl.BlockSpec((16, 16), lambda i: (0, 0)),
            pl.BlockSpec((1, 16), lambda i: (0, 0)),
            pl.BlockSpec((16, 128), lambda i: (0, 0)),
            pl.BlockSpec((EB, 128), lambda i: (0, 0)),
            pl.BlockSpec((K * D_IN, D_OUT), lambda i: (0, 0)),
        ],
        out_specs=pl.BlockSpec((BQ, D_OUT), lambda i: (i, 0)),
        out_shape=jax.ShapeDtypeStruct((N, D_OUT), jnp.float32),
    )(g, p16, qrep, kpt, ones16, kpsq, rep, mask, wflat)
    return out
